# baseline (speedup 1.0000x reference)
import functools
import math

import jax
import jax.numpy as jnp
from jax import lax
from jax.experimental import pallas as pl
from jax.experimental.pallas import tpu as pltpu
from jax.experimental.pallas import tpu_sc as plsc

NC = 2
NS = 16
NW = NC * NS
LANES = 16
CHUNK = 128


@functools.partial(jax.jit, static_argnums=(2, 3, 4))
def _lookup(xf, table, n_chunks, d, scale):
    n = NW * n_chunks * CHUNK
    mesh = plsc.VectorSubcoreMesh(core_axis_name="c", subcore_axis_name="s")

    @functools.partial(
        pl.kernel,
        mesh=mesh,
        out_type=jax.ShapeDtypeStruct((n, d), jnp.float32),
        compiler_params=pltpu.CompilerParams(use_tc_tiling_on_sc=False),
        scratch_types=[
            pltpu.VMEM((n_chunks, CHUNK), jnp.int32),
            pltpu.VMEM((CHUNK, d), jnp.float32),
            pltpu.SemaphoreType.DMA,
        ],
    )
    def k(x_hbm, table_hbm, out_hbm, idx_v, rows, sem):
        wid = lax.axis_index("s") * NC + lax.axis_index("c")
        base = wid * (n_chunks * CHUNK)
        pltpu.sync_copy(x_hbm.at[wid], idx_v)

        def chunk_body(j, carry):
            pltpu.async_copy(table_hbm.at[idx_v.at[j]], rows, sem).wait()

            def mul_body(r, c2):
                for rr in range(8):
                    for cc in range(d // LANES):
                        sl = (r * 8 + rr, pl.ds(cc * LANES, LANES))
                        rows[sl] = rows[sl] * scale
                return c2

            lax.fori_loop(0, CHUNK // 8, mul_body, 0)
            pltpu.sync_copy(rows, out_hbm.at[pl.ds(base + j * CHUNK, CHUNK)])
            return carry

        lax.fori_loop(0, n_chunks, chunk_body, 0)

    return k(xf, table)


def kernel(x, table):
    b, l = x.shape
    _, d = table.shape
    n = b * l
    per_w = n // NW
    n_chunks = per_w // CHUNK
    xf = x.reshape(NW, n_chunks, CHUNK).astype(jnp.int32)
    scale = float(math.sqrt(d))
    out = _lookup(xf, table, n_chunks, d, scale)
    return out.reshape(b, l, d)
